# speedup vs baseline: 1.0204x; 1.0204x over previous
import functools

import jax
import jax.numpy as jnp
from jax import lax
from jax.experimental import pallas as pl
from jax.experimental.pallas import tpu as pltpu

_VMEM_LIMIT = 64 * 1024 * 1024


def _conv1_kernel(x_ref, w_ref, b_ref, o_ref):
    y = lax.dot_general(x_ref[0], w_ref[...], (((0,), (0,)), ((), ())),
                        preferred_element_type=jnp.float32)
    o_ref[0] = jnp.maximum(y + b_ref[...], 0.0)


def _tail_kernel(y_ref, xr_ref, w3_ref, b3_ref, w4_ref, b4_ref, o_ref,
                 xp_ref, z_ref, *, T, H, W, P, L, PS):
    HW = H * W
    k = pl.program_id(1)

    @pl.when(k == 0)
    def _conv():
        xp_ref[...] = jnp.zeros_like(xp_ref)
        xp_ref[1:T + 1, L:L + HW, :] = y_ref[0]
        wi = lax.broadcasted_iota(jnp.int32, (HW, P), 0) % W
        ml = (wi > 0).astype(jnp.float32)
        mr = (wi < W - 1).astype(jnp.float32)
        w3 = w3_ref[...]
        for t in range(T):
            acc = None
            for kw in range(3):
                g = None
                for kt in range(3):
                    for kh in range(3):
                        d = L + (kh - 1) * W + (kw - 1)
                        v = xp_ref[t + kt, d:d + HW, :] * w3[kt, kh, kw]
                        g = v if g is None else g + v
                if kw == 0:
                    g = g * ml
                elif kw == 2:
                    g = g * mr
                acc = g if acc is None else acc + g
            z_ref[t * HW:(t + 1) * HW, :] = jnp.maximum(acc + b3_ref[...], 0.0)

    o = lax.dot_general(w4_ref[...], z_ref[...], (((0,), (1,)), ((), ())),
                        preferred_element_type=jnp.float32)
    o_ref[0] = jnp.maximum(o + b4_ref[...] + xr_ref[0], 0.0)


def _pick_schunks(S):
    for c in (8, 7, 6, 5, 4, 3, 2):
        if S % c == 0 and (S // c) % 128 == 0:
            return c
    return 1


def kernel(x, w1, a1, b1, w3, a3, b3, w4, a4, b4):
    N, CIN, T, H, W = x.shape
    P = w1.shape[1]
    COUT = w4.shape[1]
    S = T * H * W
    HW = H * W

    w1f = w1 * a1
    w3f = w3 * a3[0]
    w4f = w4 * a4
    b4t = jnp.transpose(b4)

    x3 = x.reshape(N, CIN, S)

    nsc = _pick_schunks(S)
    SC = S // nsc
    y = pl.pallas_call(
        _conv1_kernel,
        out_shape=jax.ShapeDtypeStruct((N, S, P), jnp.float32),
        grid=(N, nsc),
        in_specs=[
            pl.BlockSpec((1, CIN, SC), lambda n, s: (n, 0, s)),
            pl.BlockSpec((CIN, P), lambda n, s: (0, 0)),
            pl.BlockSpec((1, P), lambda n, s: (0, 0)),
        ],
        out_specs=pl.BlockSpec((1, SC, P), lambda n, s: (n, s, 0)),
        compiler_params=pltpu.CompilerParams(
            dimension_semantics=("parallel", "parallel"),
            vmem_limit_bytes=_VMEM_LIMIT),
    )(x3, w1f, b1)

    y4 = y.reshape(N, T, HW, P)
    KC = COUT // 128 if COUT % 128 == 0 else 1
    CK = COUT // KC
    L = ((W + 2 + 7) // 8) * 8
    PS = ((L + HW + W + 1 + 7) // 8) * 8

    tail = functools.partial(_tail_kernel, T=T, H=H, W=W, P=P, L=L, PS=PS)
    out3 = pl.pallas_call(
        tail,
        out_shape=jax.ShapeDtypeStruct((N, COUT, S), jnp.float32),
        grid=(N, KC),
        in_specs=[
            pl.BlockSpec((1, T, HW, P), lambda n, k: (n, 0, 0, 0)),
            pl.BlockSpec((1, CK, S), lambda n, k: (n, k, 0)),
            pl.BlockSpec((3, 3, 3, P), lambda n, k: (0, 0, 0, 0)),
            pl.BlockSpec((1, P), lambda n, k: (0, 0)),
            pl.BlockSpec((P, CK), lambda n, k: (0, k)),
            pl.BlockSpec((CK, 1), lambda n, k: (k, 0)),
        ],
        out_specs=pl.BlockSpec((1, CK, S), lambda n, k: (n, k, 0)),
        scratch_shapes=[
            pltpu.VMEM((T + 2, PS, P), jnp.float32),
            pltpu.VMEM((S, P), jnp.float32),
        ],
        compiler_params=pltpu.CompilerParams(
            dimension_semantics=("parallel", "arbitrary"),
            vmem_limit_bytes=_VMEM_LIMIT),
    )(y4, x3, w3f, b3, w4f, b4t)

    return out3.reshape(N, COUT, T, H, W)
